# baseline (speedup 1.0000x reference)
import functools

import jax
import jax.numpy as jnp
from jax import lax
from jax.experimental import pallas as pl
from jax.experimental.pallas import tpu as pltpu
from jax.experimental.pallas import tpu_sc as plsc

D2 = 32
F = 8
L = 16
NC = 2
NS = 16
NW = NC * NS
CHUNK = 512
SUB = 128
NSUB = CHUNK // SUB


def _emb_body(tid_hbm, base_hbm, freq_hbm, bands_hbm, out_hbm,
              idx_v, base_v, flog_v, out_v, bands_v, sem_i, sem_b, sem_f):
    n_tokens = tid_hbm.shape[0]
    tpw = n_tokens // NW
    n_chunks = tpw // CHUNK

    cid = lax.axis_index("c")
    sid = lax.axis_index("s")
    wid = sid * NC + cid
    tok_base = wid * tpw

    pltpu.sync_copy(bands_hbm, bands_v)
    band_lo = [bands_v[f, pl.ds(0, L)] for f in range(F)]
    band_hi = [bands_v[f, pl.ds(L, L)] for f in range(F)]

    def tok0_of(c):
        return pl.multiple_of(tok_base + c * CHUNK, CHUNK)

    def issue_idx(c, b):
        pltpu.async_copy(tid_hbm.at[pl.ds(tok0_of(c), CHUNK)],
                         idx_v.at[b], sem_i)

    def issue_gathers(b):
        for j in range(NSUB):
            sl = pl.ds(j * SUB, SUB)
            pltpu.async_copy(base_hbm.at[idx_v.at[b, sl]],
                             base_v.at[b, sl], sem_b)
            for f in range(F):
                pltpu.async_copy(freq_hbm.at[f].at[idx_v.at[b, sl]],
                                 flog_v.at[b, f, sl], sem_f)

    def drain_idx():
        pltpu.make_async_copy(tid_hbm.at[pl.ds(0, CHUNK)],
                              idx_v.at[0], sem_i).wait()

    def drain_gathers(b):
        for j in range(NSUB):
            sl = pl.ds(j * SUB, SUB)
            pltpu.make_async_copy(base_hbm.at[idx_v.at[b, sl]],
                                  base_v.at[b, sl], sem_b).wait()
            for f in range(F):
                pltpu.make_async_copy(freq_hbm.at[f].at[idx_v.at[b, sl]],
                                      flog_v.at[b, f, sl], sem_f).wait()

    def compute(b):
        ngrp = CHUNK // L

        def group_body(g, _):
            ls = [flog_v[b, f, pl.ds(g * L, L)] for f in range(F)]
            m01 = jnp.maximum(ls[0], ls[1])
            m23 = jnp.maximum(ls[2], ls[3])
            m45 = jnp.maximum(ls[4], ls[5])
            m67 = jnp.maximum(ls[6], ls[7])
            m = jnp.maximum(jnp.maximum(m01, m23), jnp.maximum(m45, m67))
            es = [jnp.exp(l - m) for l in ls]
            s = (((es[0] + es[1]) + (es[2] + es[3]))
                 + ((es[4] + es[5]) + (es[6] + es[7])))
            r = 1.0 / s
            ws = [e * r for e in es]
            for t in range(L):
                tok = g * L + t
                p = [ws[f][t] * band_lo[f] for f in range(F)]
                q = [ws[f][t] * band_hi[f] for f in range(F)]
                acc_lo = (((p[0] + p[1]) + (p[2] + p[3]))
                          + ((p[4] + p[5]) + (p[6] + p[7])))
                acc_hi = (((q[0] + q[1]) + (q[2] + q[3]))
                          + ((q[4] + q[5]) + (q[6] + q[7])))
                out_v[tok, pl.ds(0, L)] = base_v[b, tok, pl.ds(0, L)]
                out_v[tok, pl.ds(L, L)] = base_v[b, tok, pl.ds(L, L)]
                out_v[tok, pl.ds(2 * L, L)] = acc_lo
                out_v[tok, pl.ds(3 * L, L)] = acc_hi
            return ()

        lax.fori_loop(0, ngrp, group_body, ())

    def writeback(c):
        pltpu.sync_copy(out_v, out_hbm.at[pl.ds(tok0_of(c), CHUNK)])

    pltpu.sync_copy(tid_hbm.at[pl.ds(tok0_of(0), CHUNK)], idx_v.at[0])
    issue_gathers(0)
    issue_idx(1, 1)

    def outer_body(o, _):
        for b in range(2):
            c = o * 2 + b
            @pl.when(c + 1 < n_chunks)
            def _():
                drain_idx()
                issue_gathers(1 - b)
            drain_gathers(b)
            @pl.when(c + 2 < n_chunks)
            def _():
                issue_idx(c + 2, b)
            compute(b)
            writeback(c)
        return ()

    lax.fori_loop(0, n_chunks // 2, outer_body, ())


@jax.jit
def _emb_call(ids, base_table, freq_table, freq_bands):
    n_tokens = ids.shape[0]
    kern = pl.kernel(
        _emb_body,
        out_type=jax.ShapeDtypeStruct((n_tokens, 2 * D2), jnp.float32),
        mesh=plsc.VectorSubcoreMesh(core_axis_name="c", subcore_axis_name="s"),
        scratch_types=[
            pltpu.VMEM((2, CHUNK), jnp.int32),
            pltpu.VMEM((2, CHUNK, D2), jnp.float32),
            pltpu.VMEM((2, F, CHUNK), jnp.float32),
            pltpu.VMEM((CHUNK, 2 * D2), jnp.float32),
            pltpu.VMEM((F, D2), jnp.float32),
            pltpu.SemaphoreType.DMA,
            pltpu.SemaphoreType.DMA,
            pltpu.SemaphoreType.DMA,
        ],
        compiler_params=pltpu.CompilerParams(
            needs_layout_passes=False, use_tc_tiling_on_sc=False),
    )
    return kern(ids, base_table, freq_table, freq_bands)


def kernel(token_ids, base_table, freq_bands, freq_table):
    b, s = token_ids.shape
    ids = token_ids.astype(jnp.int32).reshape(b * s)
    out = _emb_call(ids, base_table, freq_table.T, freq_bands)
    return out.reshape(b, s, 2 * D2)


# speedup vs baseline: 1.1158x; 1.1158x over previous
import functools

import jax
import jax.numpy as jnp
from jax import lax
from jax.experimental import pallas as pl
from jax.experimental.pallas import tpu as pltpu
from jax.experimental.pallas import tpu_sc as plsc

D2 = 32
F = 8
L = 16
NC = 2
NS = 16
NW = NC * NS
CHUNK = 512
SUB = 128
NSUB = CHUNK // SUB


def _emb_body(tid_hbm, base_hbm, freq_hbm, bands_hbm, out_hbm,
              idx_v, base_v, flog_v, out_v, bands_v, sem_i, sem_b, sem_f):
    n_tokens = tid_hbm.shape[0]
    tpw = n_tokens // NW
    n_chunks = tpw // CHUNK

    cid = lax.axis_index("c")
    sid = lax.axis_index("s")
    wid = sid * NC + cid
    tok_base = wid * tpw

    pltpu.sync_copy(bands_hbm, bands_v)
    band_lo = [bands_v[f, pl.ds(0, L)] for f in range(F)]
    band_hi = [bands_v[f, pl.ds(L, L)] for f in range(F)]

    def tok0_of(c):
        return pl.multiple_of(tok_base + c * CHUNK, CHUNK)

    def issue_idx(c, b):
        pltpu.async_copy(tid_hbm.at[pl.ds(tok0_of(c), CHUNK)],
                         idx_v.at[b], sem_i)

    def issue_gathers(b):
        for j in range(NSUB):
            sl = pl.ds(j * SUB, SUB)
            pltpu.async_copy(base_hbm.at[idx_v.at[b, sl]],
                             base_v.at[b, sl], sem_b)
            pltpu.async_copy(freq_hbm.at[idx_v.at[b, sl]],
                             flog_v.at[b, sl], sem_f)

    def drain_idx():
        pltpu.make_async_copy(tid_hbm.at[pl.ds(0, CHUNK)],
                              idx_v.at[0], sem_i).wait()

    def drain_gathers(b):
        for j in range(NSUB):
            sl = pl.ds(j * SUB, SUB)
            pltpu.make_async_copy(base_hbm.at[idx_v.at[b, sl]],
                                  base_v.at[b, sl], sem_b).wait()
            pltpu.make_async_copy(freq_hbm.at[idx_v.at[b, sl]],
                                  flog_v.at[b, sl], sem_f).wait()

    def compute(b):
        ngrp = CHUNK // L

        def group_body(g, _):
            rows = g * L + lax.iota(jnp.int32, L)
            bsel = jnp.full((L,), b, jnp.int32)
            ls = [plsc.load_gather(flog_v,
                                   [bsel, rows, jnp.full((L,), f, jnp.int32)])
                  for f in range(F)]
            m01 = jnp.maximum(ls[0], ls[1])
            m23 = jnp.maximum(ls[2], ls[3])
            m45 = jnp.maximum(ls[4], ls[5])
            m67 = jnp.maximum(ls[6], ls[7])
            m = jnp.maximum(jnp.maximum(m01, m23), jnp.maximum(m45, m67))
            es = [jnp.exp(l - m) for l in ls]
            s = (((es[0] + es[1]) + (es[2] + es[3]))
                 + ((es[4] + es[5]) + (es[6] + es[7])))
            r = 1.0 / s
            ws = [e * r for e in es]
            for t in range(L):
                tok = g * L + t
                p = [ws[f][t] * band_lo[f] for f in range(F)]
                q = [ws[f][t] * band_hi[f] for f in range(F)]
                acc_lo = (((p[0] + p[1]) + (p[2] + p[3]))
                          + ((p[4] + p[5]) + (p[6] + p[7])))
                acc_hi = (((q[0] + q[1]) + (q[2] + q[3]))
                          + ((q[4] + q[5]) + (q[6] + q[7])))
                out_v[tok, pl.ds(0, L)] = base_v[b, tok, pl.ds(0, L)]
                out_v[tok, pl.ds(L, L)] = base_v[b, tok, pl.ds(L, L)]
                out_v[tok, pl.ds(2 * L, L)] = acc_lo
                out_v[tok, pl.ds(3 * L, L)] = acc_hi
            return ()

        lax.fori_loop(0, ngrp, group_body, ())

    def writeback(c):
        pltpu.sync_copy(out_v, out_hbm.at[pl.ds(tok0_of(c), CHUNK)])

    pltpu.sync_copy(tid_hbm.at[pl.ds(tok0_of(0), CHUNK)], idx_v.at[0])
    issue_gathers(0)
    issue_idx(1, 1)

    def outer_body(o, _):
        for b in range(2):
            c = o * 2 + b
            @pl.when(c + 1 < n_chunks)
            def _():
                drain_idx()
                issue_gathers(1 - b)
            drain_gathers(b)
            @pl.when(c + 2 < n_chunks)
            def _():
                issue_idx(c + 2, b)
            compute(b)
            writeback(c)
        return ()

    lax.fori_loop(0, n_chunks // 2, outer_body, ())


@jax.jit
def _emb_call(ids, base_table, freq_table, freq_bands):
    n_tokens = ids.shape[0]
    kern = pl.kernel(
        _emb_body,
        out_type=jax.ShapeDtypeStruct((n_tokens, 2 * D2), jnp.float32),
        mesh=plsc.VectorSubcoreMesh(core_axis_name="c", subcore_axis_name="s"),
        scratch_types=[
            pltpu.VMEM((2, CHUNK), jnp.int32),
            pltpu.VMEM((2, CHUNK, D2), jnp.float32),
            pltpu.VMEM((2, CHUNK, F), jnp.float32),
            pltpu.VMEM((CHUNK, 2 * D2), jnp.float32),
            pltpu.VMEM((F, D2), jnp.float32),
            pltpu.SemaphoreType.DMA,
            pltpu.SemaphoreType.DMA,
            pltpu.SemaphoreType.DMA,
        ],
        compiler_params=pltpu.CompilerParams(
            needs_layout_passes=False, use_tc_tiling_on_sc=False),
    )
    return kern(ids, base_table, freq_table, freq_bands)


def kernel(token_ids, base_table, freq_bands, freq_table):
    b, s = token_ids.shape
    ids = token_ids.astype(jnp.int32).reshape(b * s)
    out = _emb_call(ids, base_table, freq_table, freq_bands)
    return out.reshape(b, s, 2 * D2)
